# baseline (speedup 1.0000x reference)
import functools

import jax
import jax.numpy as jnp
from jax import lax
from jax.experimental import pallas as pl
from jax.experimental.pallas import tpu as pltpu
from jax.experimental.pallas import tpu_sc as plsc

_LP = 56
_BB = 16
_RP = _BB * _LP

_CHUNK = 800
_NCHUNK = 8

_INV2PI = 0.15915493667125702
_S1 = 6.283183465409584
_S3 = -41.34148025958733
_S5 = 81.59765524711814
_S7 = -76.59489967393338
_S9 = 41.26979637356374
_S11 = -12.372272029174612


def _sc_gather(rs_hbm, cs_hbm, uo_hbm, rt_hbm, ct_hbm, ut_hbm,
               r_out, c_out, u_out, idx_v, rows_v, sem):
    wid = lax.axis_index("s") * 2 + lax.axis_index("c")
    for idx_hbm, tab_hbm, out_hbm in ((rs_hbm, rt_hbm, r_out),
                                      (cs_hbm, ct_hbm, c_out),
                                      (uo_hbm, ut_hbm, u_out)):
        def chunk_body(k, carry, idx_hbm=idx_hbm, tab_hbm=tab_hbm,
                       out_hbm=out_hbm):
            base = wid * (_CHUNK * _NCHUNK) + k * _CHUNK
            pltpu.sync_copy(idx_hbm.at[pl.ds(base, _CHUNK)], idx_v)
            pltpu.async_copy(tab_hbm.at[idx_v], rows_v, sem).wait()
            pltpu.sync_copy(rows_v, out_hbm.at[pl.ds(base, _CHUNK)])
            return carry
        lax.fori_loop(0, _NCHUNK, chunk_body, 0)


def _rowmat(lhs, rhs):
    return jax.lax.dot_general(lhs, rhs, (((0,), (0,)), ((), ())),
                               preferred_element_type=jnp.float32)


def _sin2pi(f):
    f2 = f * f
    p = _S9 + f2 * _S11
    p = _S7 + f2 * p
    p = _S5 + f2 * p
    p = _S3 + f2 * p
    return f * (_S1 + f2 * p)


def _hi(x):
    b = jax.lax.bitcast_convert_type(x, jnp.uint32)
    return jax.lax.bitcast_convert_type(b & jnp.uint32(0xFFFF0000),
                                        jnp.float32)


def _tc_body(u3_ref, t5_ref, w13_ref, b1_ref, w2_ref, b2_ref,
             g_ref, be_ref, t2vw5_ref, t2vb_ref,
             uin_out, t2v_out):
    def store(ref, val):
        v = val.reshape(_BB, _LP, 128)
        ref[...] = v[:, :50, :]

    h = _rowmat(u3_ref[0], w13_ref[...]) + b1_ref[...]
    h = h * jax.nn.sigmoid(h)
    hh32 = _hi(h)
    hh = hh32.astype(jnp.bfloat16)
    hl = _hi(h - hh32).astype(jnp.bfloat16)
    zh = jnp.zeros_like(hh)
    hc = jnp.concatenate([hh, hh, hl, zh], axis=1)
    o = jnp.dot(hc, w2_ref[...], preferred_element_type=jnp.float32)
    o = o + b2_ref[...]
    mu = jnp.mean(o, axis=-1, keepdims=True)
    d = o - mu
    var = jnp.mean(d * d, axis=-1, keepdims=True)
    store(uin_out, d * jax.lax.rsqrt(var + 1e-5) * g_ref[...] + be_ref[...])

    z = _rowmat(t5_ref[0], t2vw5_ref[...]) + t2vb_ref[...]
    r = z * _INV2PI
    f = r - jnp.floor(r + 0.5)
    lane = jax.lax.broadcasted_iota(jnp.int32, z.shape, 1)
    store(t2v_out, jnp.where(lane == 0, z, _sin2pi(f)))


def _split2(x):
    h = _hi(x)
    l = x - h
    return h.astype(jnp.bfloat16), _hi(l).astype(jnp.bfloat16)


def _split3(x):
    h = _hi(x)
    r = x - h
    m = _hi(r)
    l = r - m
    return (h.astype(jnp.bfloat16), m.astype(jnp.bfloat16),
            _hi(l).astype(jnp.bfloat16))


def kernel(rs, cs, u_outs, u_ins, time_steps, r_table, c_table, u_out_table,
           W1, b1, W2, b2, ln_g, ln_b, t2v_w0, t2v_b0, t2v_W, t2v_B):
    B, L = rs.shape
    N = B * L
    H = r_table.shape[1]
    nb = B // _BB

    mesh = plsc.VectorSubcoreMesh(core_axis_name="c", subcore_axis_name="s")
    sc_call = pl.kernel(
        _sc_gather, mesh=mesh,
        out_type=[jax.ShapeDtypeStruct((B * L, H), jnp.float32)] * 3,
        scratch_types=[pltpu.VMEM((_CHUNK,), jnp.int32),
                       pltpu.VMEM((_CHUNK, H), jnp.float32),
                       pltpu.SemaphoreType.DMA],
    )
    r_lin, c_lin, uo_lin = sc_call(
        rs.astype(jnp.int32).reshape(N), cs.astype(jnp.int32).reshape(N),
        u_outs.astype(jnp.int32).reshape(N), r_table, c_table, u_out_table)
    r_feat = r_lin.reshape(B, L, H)
    c_feat = c_lin.reshape(B, L, H)
    uo_feat = uo_lin.reshape(B, L, H)

    def rows(x):
        xp = jnp.pad(x.astype(jnp.float32), ((0, 0), (0, _LP - L)))
        return xp.reshape(nb, 1, _RP)

    uh, ul = _split2(rows(u_ins))
    zr = jnp.zeros_like(uh)
    u3 = jnp.concatenate([uh, uh, ul, zr, zr, zr, zr, zr], axis=1)
    w1h, w1l = _split2(W1)
    zw1 = jnp.zeros((5, W1.shape[1]), jnp.bfloat16)
    w13 = jnp.concatenate([w1h, w1l, w1h, zw1], axis=0)

    th, tm, tl = _split3(rows(time_steps))
    t5 = jnp.concatenate([th, th, tm, tm, tl, zr, zr, zr], axis=1)
    t2v_w = jnp.concatenate([t2v_w0, t2v_W], axis=1)
    wh, wl = _split2(t2v_w)
    zw = jnp.zeros((3, t2v_w.shape[1]), jnp.bfloat16)
    t2vw5 = jnp.concatenate([wh, wl, wh, wl, wh, zw], axis=0)
    t2v_b = jnp.concatenate([t2v_b0, t2v_B], axis=0).reshape(1, H)

    w2h, w2l = _split2(W2)
    zw2 = jnp.zeros((64, W2.shape[1]), jnp.bfloat16)
    w2c = jnp.concatenate([w2h, w2l, w2h, zw2], axis=0)

    b1r = b1.reshape(1, -1)
    b2r = b2.reshape(1, H)
    gr = ln_g.reshape(1, H)
    ber = ln_b.reshape(1, H)

    spec = lambda k: pl.BlockSpec((1, k, _RP), lambda i: (i, 0, 0))
    full = lambda a: pl.BlockSpec(a.shape, lambda i: (0,) * a.ndim)
    out_spec = pl.BlockSpec((_BB, L, H), lambda i: (i, 0, 0))

    uin_feat, t2v_feat = pl.pallas_call(
        _tc_body,
        grid=(nb,),
        in_specs=[spec(8), spec(8),
                  full(w13), full(b1r), full(w2c), full(b2r),
                  full(gr), full(ber), full(t2vw5), full(t2v_b)],
        out_specs=[out_spec] * 2,
        out_shape=[jax.ShapeDtypeStruct((B, L, H), jnp.float32)] * 2,
        compiler_params=pltpu.CompilerParams(
            dimension_semantics=("arbitrary",)),
    )(u3, t5, w13, b1r, w2c, b2r, gr, ber, t2vw5, t2v_b)

    return (r_feat, c_feat, uo_feat, uin_feat, t2v_feat)


# speedup vs baseline: 14.9038x; 14.9038x over previous
import jax
import jax.numpy as jnp
from jax.experimental import pallas as pl
from jax.experimental.pallas import tpu as pltpu

_LP = 56
_BB = 32
_RP = _BB * _LP

_INV2PI = 0.15915493667125702
_S1 = 6.283183465409584
_S3 = -41.34148025958733
_S5 = 81.59765524711814
_S7 = -76.59489967393338
_S9 = 41.26979637356374
_S11 = -12.372272029174612


def _rowmat(lhs, rhs):
    return jax.lax.dot_general(lhs, rhs, (((0,), (0,)), ((), ())),
                               preferred_element_type=jnp.float32)


def _sin2pi(f):
    f2 = f * f
    p = _S9 + f2 * _S11
    p = _S7 + f2 * p
    p = _S5 + f2 * p
    p = _S3 + f2 * p
    return f * (_S1 + f2 * p)


def _body(rs_ref, cs_ref, uo_ref, u3_ref, t5_ref,
          rt_ref, ct_ref, ut_ref, w13_ref, b1_ref, w2_ref, b2_ref,
          g_ref, be_ref, t2vw5_ref, t2vb_ref,
          r_out, c_out, uo_out, uin_out, t2v_out):
    ones = jnp.ones((1, 128), dtype=jnp.bfloat16)

    def sel3(idx_ref, tab_ref):
        f = _rowmat(idx_ref[0], ones)
        t0 = tab_ref[0:1, :]
        t1 = tab_ref[1:2, :]
        t2 = tab_ref[2:3, :]
        return jnp.where(f == 0.0, t0, jnp.where(f == 1.0, t1, t2))

    def store(ref, val):
        v = val.reshape(_BB, _LP, 128)
        ref[...] = v[:, :50, :]

    store(r_out, sel3(rs_ref, rt_ref))
    store(c_out, sel3(cs_ref, ct_ref))
    fo = _rowmat(uo_ref[0], ones)
    store(uo_out, jnp.where(fo == 0.0, ut_ref[0:1, :], ut_ref[1:2, :]))

    h = _rowmat(u3_ref[0], w13_ref[...]) + b1_ref[...]
    h = h * jax.nn.sigmoid(h)
    hh32 = _hi(h)
    hh = hh32.astype(jnp.bfloat16)
    hl = _hi(h - hh32).astype(jnp.bfloat16)
    zh = jnp.zeros_like(hh)
    hc = jnp.concatenate([hh, hh, hl, zh], axis=1)
    o = jnp.dot(hc, w2_ref[...], preferred_element_type=jnp.float32)
    o = o + b2_ref[...]
    mu = jnp.mean(o, axis=-1, keepdims=True)
    d = o - mu
    var = jnp.mean(d * d, axis=-1, keepdims=True)
    store(uin_out, d * jax.lax.rsqrt(var + 1e-5) * g_ref[...] + be_ref[...])

    z = _rowmat(t5_ref[0], t2vw5_ref[...]) + t2vb_ref[...]
    r = z * _INV2PI
    f = r - jnp.floor(r + 0.5)
    lane = jax.lax.broadcasted_iota(jnp.int32, z.shape, 1)
    store(t2v_out, jnp.where(lane == 0, z, _sin2pi(f)))


def _hi(x):
    b = jax.lax.bitcast_convert_type(x, jnp.uint32)
    return jax.lax.bitcast_convert_type(b & jnp.uint32(0xFFFF0000),
                                        jnp.float32)


def _split2(x):
    h = _hi(x)
    l = x - h
    return h.astype(jnp.bfloat16), _hi(l).astype(jnp.bfloat16)


def _split3(x):
    h = _hi(x)
    r = x - h
    m = _hi(r)
    l = r - m
    return (h.astype(jnp.bfloat16), m.astype(jnp.bfloat16),
            _hi(l).astype(jnp.bfloat16))


def kernel(rs, cs, u_outs, u_ins, time_steps, r_table, c_table, u_out_table,
           W1, b1, W2, b2, ln_g, ln_b, t2v_w0, t2v_b0, t2v_W, t2v_B):
    B, L = rs.shape
    H = r_table.shape[1]
    nb = B // _BB

    def rows(x):
        xp = jnp.pad(x.astype(jnp.float32), ((0, 0), (0, _LP - L)))
        return xp.reshape(nb, 1, _RP)

    rs2 = rows(rs).astype(jnp.bfloat16)
    cs2 = rows(cs).astype(jnp.bfloat16)
    uo2 = rows(u_outs).astype(jnp.bfloat16)

    uh, ul = _split2(rows(u_ins))
    zr = jnp.zeros_like(uh)
    u3 = jnp.concatenate([uh, uh, ul, zr, zr, zr, zr, zr], axis=1)
    w1h, w1l = _split2(W1)
    zw1 = jnp.zeros((5, W1.shape[1]), jnp.bfloat16)
    w13 = jnp.concatenate([w1h, w1l, w1h, zw1], axis=0)

    th, tm, tl = _split3(rows(time_steps))
    t5 = jnp.concatenate([th, th, tm, tm, tl, zr, zr, zr], axis=1)
    t2v_w = jnp.concatenate([t2v_w0, t2v_W], axis=1)
    wh, wl = _split2(t2v_w)
    zw = jnp.zeros((3, t2v_w.shape[1]), jnp.bfloat16)
    t2vw5 = jnp.concatenate([wh, wl, wh, wl, wh, zw], axis=0)
    t2v_b = jnp.concatenate([t2v_b0, t2v_B], axis=0).reshape(1, H)

    w2h, w2l = _split2(W2)
    zw2 = jnp.zeros((64, W2.shape[1]), jnp.bfloat16)
    w2c = jnp.concatenate([w2h, w2l, w2h, zw2], axis=0)

    b1r = b1.reshape(1, -1)
    b2r = b2.reshape(1, H)
    gr = ln_g.reshape(1, H)
    ber = ln_b.reshape(1, H)

    spec = lambda k: pl.BlockSpec((1, k, _RP), lambda i: (i, 0, 0))
    full = lambda a: pl.BlockSpec(a.shape, lambda i: (0,) * a.ndim)
    out_spec = pl.BlockSpec((_BB, L, H), lambda i: (i, 0, 0))

    outs = pl.pallas_call(
        _body,
        grid=(nb,),
        in_specs=[spec(1), spec(1), spec(1), spec(8), spec(8),
                  full(r_table), full(c_table), full(u_out_table),
                  full(w13), full(b1r), full(w2c), full(b2r),
                  full(gr), full(ber), full(t2vw5), full(t2v_b)],
        out_specs=[out_spec] * 5,
        out_shape=[jax.ShapeDtypeStruct((B, L, H), jnp.float32)] * 5,
        compiler_params=pltpu.CompilerParams(
            dimension_semantics=("arbitrary",)),
    )(rs2, cs2, uo2, u3, t5, r_table, c_table, u_out_table,
      w13, b1r, w2c, b2r, gr, ber, t2vw5, t2v_b)

    return tuple(outs)


# speedup vs baseline: 14.9134x; 1.0006x over previous
import jax
import jax.numpy as jnp
from jax.experimental import pallas as pl
from jax.experimental.pallas import tpu as pltpu

_LP = 56
_BB = 64
_RP = _BB * _LP

_INV2PI = 0.15915493667125702
_S1 = 6.283183465409584
_S3 = -41.34148025958733
_S5 = 81.59765524711814
_S7 = -76.59489967393338
_S9 = 41.26979637356374
_S11 = -12.372272029174612


def _rowmat(lhs, rhs):
    return jax.lax.dot_general(lhs, rhs, (((0,), (0,)), ((), ())),
                               preferred_element_type=jnp.float32)


def _sin2pi(f):
    f2 = f * f
    p = _S9 + f2 * _S11
    p = _S7 + f2 * p
    p = _S5 + f2 * p
    p = _S3 + f2 * p
    return f * (_S1 + f2 * p)


def _body(rs_ref, cs_ref, uo_ref, u3_ref, t5_ref,
          rt_ref, ct_ref, ut_ref, w13_ref, b1_ref, w2_ref, b2_ref,
          g_ref, be_ref, t2vw5_ref, t2vb_ref,
          r_out, c_out, uo_out, uin_out, t2v_out):
    ones = jnp.ones((1, 128), dtype=jnp.bfloat16)

    def sel3(idx_ref, tab_ref):
        f = _rowmat(idx_ref[0], ones)
        t0 = tab_ref[0:1, :]
        t1 = tab_ref[1:2, :]
        t2 = tab_ref[2:3, :]
        return jnp.where(f == 0.0, t0, jnp.where(f == 1.0, t1, t2))

    def store(ref, val):
        v = val.reshape(_BB, _LP, 128)
        ref[...] = v[:, :50, :]

    store(r_out, sel3(rs_ref, rt_ref))
    store(c_out, sel3(cs_ref, ct_ref))
    fo = _rowmat(uo_ref[0], ones)
    store(uo_out, jnp.where(fo == 0.0, ut_ref[0:1, :], ut_ref[1:2, :]))

    h = _rowmat(u3_ref[0], w13_ref[...]) + b1_ref[...]
    h = h * jax.nn.sigmoid(h)
    hh32 = _hi(h)
    hh = hh32.astype(jnp.bfloat16)
    hl = _hi(h - hh32).astype(jnp.bfloat16)
    zh = jnp.zeros_like(hh)
    hc = jnp.concatenate([hh, hh, hl, zh], axis=1)
    o = jnp.dot(hc, w2_ref[...], preferred_element_type=jnp.float32)
    o = o + b2_ref[...]
    mu = jnp.mean(o, axis=-1, keepdims=True)
    d = o - mu
    var = jnp.mean(d * d, axis=-1, keepdims=True)
    store(uin_out, d * jax.lax.rsqrt(var + 1e-5) * g_ref[...] + be_ref[...])

    z = _rowmat(t5_ref[0], t2vw5_ref[...]) + t2vb_ref[...]
    r = z * _INV2PI
    f = r - jnp.floor(r + 0.5)
    lane = jax.lax.broadcasted_iota(jnp.int32, z.shape, 1)
    store(t2v_out, jnp.where(lane == 0, z, _sin2pi(f)))


def _hi(x):
    b = jax.lax.bitcast_convert_type(x, jnp.uint32)
    return jax.lax.bitcast_convert_type(b & jnp.uint32(0xFFFF0000),
                                        jnp.float32)


def _split2(x):
    h = _hi(x)
    l = x - h
    return h.astype(jnp.bfloat16), _hi(l).astype(jnp.bfloat16)


def _split3(x):
    h = _hi(x)
    r = x - h
    m = _hi(r)
    l = r - m
    return (h.astype(jnp.bfloat16), m.astype(jnp.bfloat16),
            _hi(l).astype(jnp.bfloat16))


def kernel(rs, cs, u_outs, u_ins, time_steps, r_table, c_table, u_out_table,
           W1, b1, W2, b2, ln_g, ln_b, t2v_w0, t2v_b0, t2v_W, t2v_B):
    B, L = rs.shape
    H = r_table.shape[1]
    nb = B // _BB

    def rows(x):
        xp = jnp.pad(x.astype(jnp.float32), ((0, 0), (0, _LP - L)))
        return xp.reshape(nb, 1, _RP)

    rs2 = rows(rs).astype(jnp.bfloat16)
    cs2 = rows(cs).astype(jnp.bfloat16)
    uo2 = rows(u_outs).astype(jnp.bfloat16)

    uh, ul = _split2(rows(u_ins))
    zr = jnp.zeros_like(uh)
    u3 = jnp.concatenate([uh, uh, ul, zr, zr, zr, zr, zr], axis=1)
    w1h, w1l = _split2(W1)
    zw1 = jnp.zeros((5, W1.shape[1]), jnp.bfloat16)
    w13 = jnp.concatenate([w1h, w1l, w1h, zw1], axis=0)

    th, tm, tl = _split3(rows(time_steps))
    t5 = jnp.concatenate([th, th, tm, tm, tl, zr, zr, zr], axis=1)
    t2v_w = jnp.concatenate([t2v_w0, t2v_W], axis=1)
    wh, wl = _split2(t2v_w)
    zw = jnp.zeros((3, t2v_w.shape[1]), jnp.bfloat16)
    t2vw5 = jnp.concatenate([wh, wl, wh, wl, wh, zw], axis=0)
    t2v_b = jnp.concatenate([t2v_b0, t2v_B], axis=0).reshape(1, H)

    w2h, w2l = _split2(W2)
    zw2 = jnp.zeros((64, W2.shape[1]), jnp.bfloat16)
    w2c = jnp.concatenate([w2h, w2l, w2h, zw2], axis=0)

    b1r = b1.reshape(1, -1)
    b2r = b2.reshape(1, H)
    gr = ln_g.reshape(1, H)
    ber = ln_b.reshape(1, H)

    spec = lambda k: pl.BlockSpec((1, k, _RP), lambda i: (i, 0, 0))
    full = lambda a: pl.BlockSpec(a.shape, lambda i: (0,) * a.ndim)
    out_spec = pl.BlockSpec((_BB, L, H), lambda i: (i, 0, 0))

    outs = pl.pallas_call(
        _body,
        grid=(nb,),
        in_specs=[spec(1), spec(1), spec(1), spec(8), spec(8),
                  full(r_table), full(c_table), full(u_out_table),
                  full(w13), full(b1r), full(w2c), full(b2r),
                  full(gr), full(ber), full(t2vw5), full(t2v_b)],
        out_specs=[out_spec] * 5,
        out_shape=[jax.ShapeDtypeStruct((B, L, H), jnp.float32)] * 5,
        compiler_params=pltpu.CompilerParams(
            dimension_semantics=("arbitrary",)),
    )(rs2, cs2, uo2, u3, t5, r_table, c_table, u_out_table,
      w13, b1r, w2c, b2r, gr, ber, t2vw5, t2v_b)

    return tuple(outs)


# speedup vs baseline: 14.9313x; 1.0012x over previous
import jax
import jax.numpy as jnp
from jax.experimental import pallas as pl
from jax.experimental.pallas import tpu as pltpu

_LP = 56
_BB = 32
_RP = _BB * _LP

_INV2PI = 0.15915493667125702
_S1 = 6.283183465409584
_S3 = -41.34148025958733
_S5 = 81.59765524711814
_S7 = -76.59489967393338
_S9 = 41.26979637356374
_S11 = -12.372272029174612


def _rowmat(lhs, rhs):
    return jax.lax.dot_general(lhs, rhs, (((0,), (0,)), ((), ())),
                               preferred_element_type=jnp.float32)


def _sin2pi(f):
    f2 = f * f
    p = _S9 + f2 * _S11
    p = _S7 + f2 * p
    p = _S5 + f2 * p
    p = _S3 + f2 * p
    return f * (_S1 + f2 * p)


def _body(rs_ref, cs_ref, uo_ref, u3_ref, t5_ref,
          rt_ref, ct_ref, ut_ref, w13_ref, b1_ref, w2_ref, b2_ref,
          g_ref, be_ref, t2vw5_ref, t2vb_ref,
          r_out, c_out, uo_out, uin_out, t2v_out):
    ones = jnp.ones((1, 128), dtype=jnp.bfloat16)

    def sel3(idx_ref, tab_ref):
        f = _rowmat(idx_ref[0], ones)
        t0 = tab_ref[0:1, :]
        t1 = tab_ref[1:2, :]
        t2 = tab_ref[2:3, :]
        return jnp.where(f == 0.0, t0, jnp.where(f == 1.0, t1, t2))

    def store(ref, val):
        v = val.reshape(_BB, _LP, 128)
        ref[...] = v[:, :50, :]

    store(r_out, sel3(rs_ref, rt_ref))
    store(c_out, sel3(cs_ref, ct_ref))
    fo = _rowmat(uo_ref[0], ones)
    store(uo_out, jnp.where(fo == 0.0, ut_ref[0:1, :], ut_ref[1:2, :]))

    h = _rowmat(u3_ref[0], w13_ref[...]) + b1_ref[...]
    h = h * jax.nn.sigmoid(h)
    hh32 = _hi(h)
    hh = hh32.astype(jnp.bfloat16)
    hl = _hi(h - hh32).astype(jnp.bfloat16)
    zh = jnp.zeros_like(hh)
    hc = jnp.concatenate([hh, hh, hl, zh], axis=1)
    o = jnp.dot(hc, w2_ref[...], preferred_element_type=jnp.float32)
    o = o + b2_ref[...]
    mu = jnp.mean(o, axis=-1, keepdims=True)
    d = o - mu
    var = jnp.mean(d * d, axis=-1, keepdims=True)
    store(uin_out, d * jax.lax.rsqrt(var + 1e-5) * g_ref[...] + be_ref[...])

    z = _rowmat(t5_ref[0], t2vw5_ref[...]) + t2vb_ref[...]
    r = z * _INV2PI
    f = r - jnp.floor(r + 0.5)
    lane = jax.lax.broadcasted_iota(jnp.int32, z.shape, 1)
    store(t2v_out, jnp.where(lane == 0, z, _sin2pi(f)))


def _hi(x):
    b = jax.lax.bitcast_convert_type(x, jnp.uint32)
    return jax.lax.bitcast_convert_type(b & jnp.uint32(0xFFFF0000),
                                        jnp.float32)


def _split2(x):
    h = _hi(x)
    l = x - h
    return h.astype(jnp.bfloat16), _hi(l).astype(jnp.bfloat16)


def _split3(x):
    h = _hi(x)
    r = x - h
    m = _hi(r)
    l = r - m
    return (h.astype(jnp.bfloat16), m.astype(jnp.bfloat16),
            _hi(l).astype(jnp.bfloat16))


def kernel(rs, cs, u_outs, u_ins, time_steps, r_table, c_table, u_out_table,
           W1, b1, W2, b2, ln_g, ln_b, t2v_w0, t2v_b0, t2v_W, t2v_B):
    B, L = rs.shape
    H = r_table.shape[1]
    nb = B // _BB

    def rows(x):
        xp = jnp.pad(x.astype(jnp.float32), ((0, 0), (0, _LP - L)))
        return xp.reshape(nb, 1, _RP)

    rs2 = rows(rs).astype(jnp.bfloat16)
    cs2 = rows(cs).astype(jnp.bfloat16)
    uo2 = rows(u_outs).astype(jnp.bfloat16)

    uh, ul = _split2(rows(u_ins))
    zr = jnp.zeros_like(uh)
    u3 = jnp.concatenate([uh, uh, ul, zr, zr, zr, zr, zr], axis=1)
    w1h, w1l = _split2(W1)
    zw1 = jnp.zeros((5, W1.shape[1]), jnp.bfloat16)
    w13 = jnp.concatenate([w1h, w1l, w1h, zw1], axis=0)

    th, tm, tl = _split3(rows(time_steps))
    t5 = jnp.concatenate([th, th, tm, tm, tl, zr, zr, zr], axis=1)
    t2v_w = jnp.concatenate([t2v_w0, t2v_W], axis=1)
    wh, wl = _split2(t2v_w)
    zw = jnp.zeros((3, t2v_w.shape[1]), jnp.bfloat16)
    t2vw5 = jnp.concatenate([wh, wl, wh, wl, wh, zw], axis=0)
    t2v_b = jnp.concatenate([t2v_b0, t2v_B], axis=0).reshape(1, H)

    w2h, w2l = _split2(W2)
    zw2 = jnp.zeros((64, W2.shape[1]), jnp.bfloat16)
    w2c = jnp.concatenate([w2h, w2l, w2h, zw2], axis=0)

    b1r = b1.reshape(1, -1)
    b2r = b2.reshape(1, H)
    gr = ln_g.reshape(1, H)
    ber = ln_b.reshape(1, H)

    spec = lambda k: pl.BlockSpec((1, k, _RP), lambda i: (i, 0, 0))
    full = lambda a: pl.BlockSpec(a.shape, lambda i: (0,) * a.ndim)
    out_spec = pl.BlockSpec((_BB, L, H), lambda i: (i, 0, 0))

    outs = pl.pallas_call(
        _body,
        grid=(nb,),
        in_specs=[spec(1), spec(1), spec(1), spec(8), spec(8),
                  full(r_table), full(c_table), full(u_out_table),
                  full(w13), full(b1r), full(w2c), full(b2r),
                  full(gr), full(ber), full(t2vw5), full(t2v_b)],
        out_specs=[out_spec] * 5,
        out_shape=[jax.ShapeDtypeStruct((B, L, H), jnp.float32)] * 5,
        compiler_params=pltpu.CompilerParams(
            dimension_semantics=("arbitrary",)),
    )(rs2, cs2, uo2, u3, t5, r_table, c_table, u_out_table,
      w13, b1r, w2c, b2r, gr, ber, t2vw5, t2v_b)

    return tuple(outs)
